# baseline (speedup 1.0000x reference)
import jax
import jax.numpy as jnp
from jax.experimental import pallas as pl
from jax.experimental.pallas import tpu as pltpu

B, S, D = 4, 2048, 1024
CH_R = 512
CPB = S // CH_R
N = B * CPB
NBUF = 4
LOOKAHEAD = NBUF - 1


def _body(x_hbm, p_ref, o_hbm, in_bufs, out_bufs, in_sems, out_sems):
    def in_copy(c):
        slot = jax.lax.rem(c, NBUF)
        b = c // CPB
        r0 = jax.lax.rem(c, CPB) * CH_R
        return pltpu.make_async_copy(
            x_hbm.at[b, pl.ds(r0, CH_R), :], in_bufs.at[slot], in_sems.at[slot]
        )

    def out_copy(c):
        slot = jax.lax.rem(c, NBUF)
        b = c // CPB
        r0 = jax.lax.rem(c, CPB) * CH_R
        return pltpu.make_async_copy(
            out_bufs.at[slot], o_hbm.at[b, pl.ds(r0, CH_R), :], out_sems.at[slot]
        )

    for c in range(LOOKAHEAD):
        in_copy(c).start()

    def step(i, _):
        @pl.when(i + LOOKAHEAD < N)
        def _():
            in_copy(i + LOOKAHEAD).start()

        in_copy(i).wait()

        @pl.when(i >= NBUF)
        def _():
            out_copy(i - NBUF).wait()

        slot = jax.lax.rem(i, NBUF)
        r0 = jax.lax.rem(i, CPB) * CH_R
        p = p_ref[pl.ds(r0, CH_R), :]
        out_bufs[slot] = in_bufs[slot] + p
        out_copy(i).start()
        return 0

    jax.lax.fori_loop(0, N, step, 0)
    for j in range(NBUF):
        out_copy(N - NBUF + j).wait()


def kernel(inputs, pos_weight):
    return pl.pallas_call(
        _body,
        in_specs=[
            pl.BlockSpec(memory_space=pl.ANY),
            pl.BlockSpec((S, 1), lambda: (0, 0)),
        ],
        out_specs=pl.BlockSpec(memory_space=pl.ANY),
        out_shape=jax.ShapeDtypeStruct((B, S, D), jnp.float32),
        scratch_shapes=[
            pltpu.VMEM((NBUF, CH_R, D), jnp.float32),
            pltpu.VMEM((NBUF, CH_R, D), jnp.float32),
            pltpu.SemaphoreType.DMA((NBUF,)),
            pltpu.SemaphoreType.DMA((NBUF,)),
        ],
        compiler_params=pltpu.CompilerParams(
            vmem_limit_bytes=100 * 1024 * 1024,
        ),
    )(inputs, pos_weight)


# speedup vs baseline: 1.0082x; 1.0082x over previous
import jax
import jax.numpy as jnp
from jax.experimental import pallas as pl
from jax.experimental.pallas import tpu as pltpu

B, S, D = 4, 2048, 1024
CH_R = 1024
CPB = S // CH_R
N = B * CPB
NBUF = 4
LOOKAHEAD = NBUF - 1


def _body(x_hbm, p_ref, o_hbm, in_bufs, out_bufs, in_sems, out_sems):
    def in_copy(c):
        slot = jax.lax.rem(c, NBUF)
        b = c // CPB
        r0 = jax.lax.rem(c, CPB) * CH_R
        return pltpu.make_async_copy(
            x_hbm.at[b, pl.ds(r0, CH_R), :], in_bufs.at[slot], in_sems.at[slot]
        )

    def out_copy(c):
        slot = jax.lax.rem(c, NBUF)
        b = c // CPB
        r0 = jax.lax.rem(c, CPB) * CH_R
        return pltpu.make_async_copy(
            out_bufs.at[slot], o_hbm.at[b, pl.ds(r0, CH_R), :], out_sems.at[slot]
        )

    for c in range(LOOKAHEAD):
        in_copy(c).start()

    def step(i, _):
        @pl.when(i + LOOKAHEAD < N)
        def _():
            in_copy(i + LOOKAHEAD).start()

        in_copy(i).wait()

        @pl.when(i >= NBUF)
        def _():
            out_copy(i - NBUF).wait()

        slot = jax.lax.rem(i, NBUF)
        r0 = jax.lax.rem(i, CPB) * CH_R
        p = p_ref[pl.ds(r0, CH_R), :]
        out_bufs[slot] = in_bufs[slot] + p
        out_copy(i).start()
        return 0

    jax.lax.fori_loop(0, N, step, 0)
    for j in range(NBUF):
        out_copy(N - NBUF + j).wait()


def kernel(inputs, pos_weight):
    return pl.pallas_call(
        _body,
        in_specs=[
            pl.BlockSpec(memory_space=pl.ANY),
            pl.BlockSpec((S, 1), lambda: (0, 0)),
        ],
        out_specs=pl.BlockSpec(memory_space=pl.ANY),
        out_shape=jax.ShapeDtypeStruct((B, S, D), jnp.float32),
        scratch_shapes=[
            pltpu.VMEM((NBUF, CH_R, D), jnp.float32),
            pltpu.VMEM((NBUF, CH_R, D), jnp.float32),
            pltpu.SemaphoreType.DMA((NBUF,)),
            pltpu.SemaphoreType.DMA((NBUF,)),
        ],
        compiler_params=pltpu.CompilerParams(
            vmem_limit_bytes=100 * 1024 * 1024,
        ),
    )(inputs, pos_weight)
